# baseline (speedup 1.0000x reference)
import functools

import jax
import jax.numpy as jnp
from jax import lax
from jax.experimental import pallas as pl
from jax.experimental.pallas import tpu as pltpu
from jax.experimental.pallas import tpu_sc as plsc

D = 1024
B = 4 * 8192
NC, NS = 2, 16
NW = NC * NS
B_PER_W = B // NW
L = 16


@functools.partial(
    pl.kernel,
    out_type=jax.ShapeDtypeStruct((B, D), jnp.float32),
    mesh=plsc.VectorSubcoreMesh(core_axis_name="c", subcore_axis_name="s"),
    compiler_params=pltpu.CompilerParams(
        use_tc_tiling_on_sc=False, needs_layout_passes=False
    ),
    scratch_types=[
        pltpu.VMEM((64, D), jnp.float32),
        pltpu.VMEM((64, D), jnp.float32),
    ],
)
def _probe(table_hbm, idx_hbm, out_hbm, src_v, buf):
    wid = lax.axis_index("c") * NS + lax.axis_index("s")
    base = wid * B_PER_W
    pltpu.sync_copy(table_hbm.at[pl.ds(0, 64)], src_v)

    def body(i, carry):
        def inner(j, c2):
            r = j // 8
            cc = (j % 8) * 128
            for u in range(8):
                w = src_v[r, pl.ds(cc + u * L, L)]
                buf[r, pl.ds(cc + u * L, L)] = w
            return c2

        lax.fori_loop(0, 64 * 8, inner, 0)
        return carry

    lax.fori_loop(0, 16, body, 0)
    pltpu.sync_copy(buf, out_hbm.at[pl.ds(base, 64)])


def kernel(x, table):
    idx = x.reshape(-1).astype(jnp.int32)
    out = _probe(table, idx)
    return out.reshape(x.shape + (table.shape[1],))


# speedup vs baseline: 1.4269x; 1.4269x over previous
import functools

import jax
import jax.numpy as jnp
from jax import lax
from jax.experimental import pallas as pl
from jax.experimental.pallas import tpu as pltpu
from jax.experimental.pallas import tpu_sc as plsc

D = 1024
B = 4 * 8192
NC, NS = 2, 16
NW = NC * NS
B_PER_W = B // NW
R = 16
NBUF = 4
NCHUNK = B_PER_W // R


@functools.partial(
    pl.kernel,
    out_type=jax.ShapeDtypeStruct((B, D), jnp.float32),
    mesh=plsc.VectorSubcoreMesh(core_axis_name="c", subcore_axis_name="s"),
    scratch_types=[
        pltpu.VMEM((B_PER_W,), jnp.int32),
        pltpu.VMEM((NBUF, R, D), jnp.float32),
        pltpu.SemaphoreType.DMA,
        pltpu.SemaphoreType.DMA,
        pltpu.SemaphoreType.DMA,
        pltpu.SemaphoreType.DMA,
    ],
)
def _embed_lookup(table_hbm, idx_hbm, out_hbm, idx_v, bufs, g0, g1, g2, g3):
    wid = lax.axis_index("c") * NS + lax.axis_index("s")
    base = wid * B_PER_W
    pltpu.sync_copy(idx_hbm.at[pl.ds(base, B_PER_W)], idx_v)

    gsems = (g0, g1, g2, g3)

    def gather_start(c, b):
        pltpu.async_copy(
            table_hbm.at[idx_v.at[pl.ds(c * R, R)]], bufs.at[b], gsems[b]
        )

    def gather_wait(b):
        pltpu.make_async_copy(
            table_hbm.at[pl.ds(0, R)], bufs.at[b], gsems[b]
        ).wait()

    def store(c, b):
        pltpu.sync_copy(bufs.at[b], out_hbm.at[pl.ds(base + c * R, R)])

    for b in range(NBUF):
        gather_start(b, b)

    def body(i, carry):
        c = i * NBUF
        for b in range(NBUF):
            gather_wait(b)
            store(c + b, b)
            gather_start(c + b + NBUF, b)
        return carry

    lax.fori_loop(0, NCHUNK // NBUF - 1, body, 0)
    for b in range(NBUF):
        gather_wait(b)
        store(NCHUNK - NBUF + b, b)


def kernel(x, table):
    idx = x.reshape(-1).astype(jnp.int32)
    out = _embed_lookup(table, idx)
    return out.reshape(x.shape + (table.shape[1],))
